# baseline (speedup 1.0000x reference)
import functools

import jax
import jax.numpy as jnp
from jax import lax
from jax.experimental import pallas as pl
from jax.experimental.pallas import tpu as pltpu
from jax.experimental.pallas import tpu_sc as plsc

BATCH = 16384
VOCAB = 100000
EMB = 128
CTX = 6
OUT = 91

NUM_WORKERS = 32
CHUNK = 384
NCH = 4
BCH = BATCH // NCH
RWS = BCH * CTX
RPW = RWS // NUM_WORKERS


def _gather_body(table_hbm, idx_hbm, out_hbm, idx_v, rows_a, rows_b, sem_a, sem_b):
    nchunk = RPW // CHUNK
    wid = lax.axis_index("s") * 2 + lax.axis_index("c")
    base = wid * RPW
    pltpu.sync_copy(idx_hbm.at[pl.ds(base, RPW)], idx_v)
    bufs = (rows_a, rows_b)
    sems = (sem_a, sem_b)
    cps = [None, None]
    cps[0] = pltpu.async_copy(table_hbm.at[idx_v.at[pl.ds(0, CHUNK)]], bufs[0], sems[0])
    for c in range(nchunk):
        nxt = (c + 1) % 2
        if c + 1 < nchunk:
            cps[nxt] = pltpu.async_copy(
                table_hbm.at[idx_v.at[pl.ds((c + 1) * CHUNK, CHUNK)]], bufs[nxt], sems[nxt])
        cps[c % 2].wait()
        pltpu.sync_copy(bufs[c % 2], out_hbm.at[pl.ds(base + c * CHUNK, CHUNK)])


_gather = pl.kernel(
    _gather_body,
    out_type=jax.ShapeDtypeStruct((RWS, EMB), jnp.float32),
    mesh=plsc.VectorSubcoreMesh(core_axis_name="c", subcore_axis_name="s"),
    scratch_types=[
        pltpu.VMEM((RPW,), jnp.int32),
        pltpu.VMEM((CHUNK, EMB), jnp.float32),
        pltpu.VMEM((CHUNK, EMB), jnp.float32),
        pltpu.SemaphoreType.DMA,
        pltpu.SemaphoreType.DMA,
    ],
)


def _mlp_body(x_ref, w1_ref, b1_ref, w2_ref, b2_ref, out_ref):
    x = x_ref[...]
    h = jnp.maximum(
        jax.lax.dot_general(x, w1_ref[...], (((1,), (0,)), ((), ())),
                            preferred_element_type=jnp.float32) + b1_ref[...],
        0.0)
    logits = jax.lax.dot_general(h, w2_ref[...], (((1,), (0,)), ((), ())),
                                 preferred_element_type=jnp.float32) + b2_ref[...]
    m = jnp.max(logits, axis=1, keepdims=True)
    s = logits - m
    lse = jnp.log(jnp.sum(jnp.exp(s), axis=1, keepdims=True))
    out_ref[...] = s - lse


BLOCK_B = 1024


def _mlp(x, W1, b1, W2, b2):
    grid = (BCH // BLOCK_B,)
    return pl.pallas_call(
        _mlp_body,
        grid=grid,
        in_specs=[
            pl.BlockSpec((BLOCK_B, CTX * EMB), lambda i: (i, 0)),
            pl.BlockSpec((CTX * EMB, EMB), lambda i: (0, 0)),
            pl.BlockSpec((1, EMB), lambda i: (0, 0)),
            pl.BlockSpec((EMB, OUT), lambda i: (0, 0)),
            pl.BlockSpec((1, OUT), lambda i: (0, 0)),
        ],
        out_specs=pl.BlockSpec((BLOCK_B, OUT), lambda i: (i, 0)),
        out_shape=jax.ShapeDtypeStruct((BCH, OUT), jnp.float32),
    )(x, W1, b1, W2, b2)


@jax.jit
def kernel(inputs, emb, W1, b1, W2, b2):
    idx = inputs.reshape(-1)
    b1r = b1.reshape(1, EMB)
    b2r = b2.reshape(1, OUT)
    outs = []
    for i in range(NCH):
        g = _gather(emb, idx[i * RWS:(i + 1) * RWS])
        x = g.reshape(BCH, CTX * EMB)
        outs.append(_mlp(x, W1, b1r, W2, b2r))
    return jnp.concatenate(outs, axis=0)


# speedup vs baseline: 1.6766x; 1.6766x over previous
import functools

import jax
import jax.numpy as jnp
from jax import lax
from jax.experimental import pallas as pl
from jax.experimental.pallas import tpu as pltpu
from jax.experimental.pallas import tpu_sc as plsc

BATCH = 16384
VOCAB = 100000
EMB = 128
CTX = 6
OUT = 91

NUM_WORKERS = 32
BSPAN = BATCH // NUM_WORKERS
CHUNK = 256
SUB = BSPAN // CHUNK


def _gather_body(table_hbm, idxT_hbm, out_hbm, idx_v, rows_a, rows_b, sem_a, sem_b):
    wid = lax.axis_index("s") * 2 + lax.axis_index("c")
    b0 = wid * BSPAN
    for t in range(CTX):
        pltpu.sync_copy(idxT_hbm.at[t, pl.ds(b0, BSPAN)],
                        idx_v.at[pl.ds(t * BSPAN, BSPAN)])
    bufs = (rows_a, rows_b)
    sems = (sem_a, sem_b)
    nsteps = CTX * SUB
    cps = [None, None]

    def src(step):
        return table_hbm.at[idx_v.at[pl.ds(step * CHUNK, CHUNK)]]

    def dst(step):
        t, s = divmod(step, SUB)
        return out_hbm.at[pl.ds(t * BATCH + b0 + s * CHUNK, CHUNK)]

    cps[0] = pltpu.async_copy(src(0), bufs[0], sems[0])
    for c in range(nsteps):
        nxt = (c + 1) % 2
        if c + 1 < nsteps:
            cps[nxt] = pltpu.async_copy(src(c + 1), bufs[nxt], sems[nxt])
        cps[c % 2].wait()
        pltpu.sync_copy(bufs[c % 2], dst(c))


_gather = pl.kernel(
    _gather_body,
    out_type=jax.ShapeDtypeStruct((CTX * BATCH, EMB), jnp.float32),
    mesh=plsc.VectorSubcoreMesh(core_axis_name="c", subcore_axis_name="s"),
    scratch_types=[
        pltpu.VMEM((CTX * BSPAN,), jnp.int32),
        pltpu.VMEM((CHUNK, EMB), jnp.float32),
        pltpu.VMEM((CHUNK, EMB), jnp.float32),
        pltpu.SemaphoreType.DMA,
        pltpu.SemaphoreType.DMA,
    ],
)


def _mlp_body(x_ref, w1_ref, b1_ref, w2_ref, b2_ref, out_ref):
    acc = b1_ref[...].astype(jnp.float32)
    h = jnp.broadcast_to(acc, (x_ref.shape[1], EMB))
    for j in range(CTX):
        h = h + jax.lax.dot_general(
            x_ref[j], w1_ref[j], (((1,), (0,)), ((), ())),
            preferred_element_type=jnp.float32)
    h = jnp.maximum(h, 0.0)
    logits = jax.lax.dot_general(h, w2_ref[...], (((1,), (0,)), ((), ())),
                                 preferred_element_type=jnp.float32) + b2_ref[...]
    m = jnp.max(logits, axis=1, keepdims=True)
    s = logits - m
    lse = jnp.log(jnp.sum(jnp.exp(s), axis=1, keepdims=True))
    out_ref[...] = s - lse


BLOCK_B = 1024


def _mlp(x3, W1r, b1, W2, b2):
    grid = (BATCH // BLOCK_B,)
    return pl.pallas_call(
        _mlp_body,
        grid=grid,
        in_specs=[
            pl.BlockSpec((CTX, BLOCK_B, EMB), lambda i: (0, i, 0)),
            pl.BlockSpec((CTX, EMB, EMB), lambda i: (0, 0, 0)),
            pl.BlockSpec((1, EMB), lambda i: (0, 0)),
            pl.BlockSpec((EMB, OUT), lambda i: (0, 0)),
            pl.BlockSpec((1, OUT), lambda i: (0, 0)),
        ],
        out_specs=pl.BlockSpec((BLOCK_B, OUT), lambda i: (i, 0)),
        out_shape=jax.ShapeDtypeStruct((BATCH, OUT), jnp.float32),
    )(x3, W1r, b1, W2, b2)


@jax.jit
def kernel(inputs, emb, W1, b1, W2, b2):
    idxT = jnp.transpose(inputs)
    gathered = _gather(emb, idxT)
    x3 = gathered.reshape(CTX, BATCH, EMB)
    W1r = W1.reshape(CTX, EMB, EMB)
    return _mlp(x3, W1r, b1.reshape(1, EMB), W2, b2.reshape(1, OUT))
